# baseline (speedup 1.0000x reference)
import functools

import jax
import jax.numpy as jnp
from jax import lax
from jax.experimental import pallas as pl
from jax.experimental.pallas import tpu as pltpu
from jax.experimental.pallas import tpu_sc as plsc

_N = 10000
_E = 320000
_IN = 200
_H = 5
_C = 10
_HC = 50
_G = 16
_D = 64
_DD = 16
_OFF = 14
_NC = 2
_NS = 16
_NW = _NC * _NS
_EPT = _E // _NW
_B = 100
_NCH = _EPT // _B
_STRIPE = 1000
_NSTRIPE = _N // _STRIPE


def _node_tables_body(x_ref, w_ref, as_ref, ad_ref, st_ref, dt_ref):
  xb = x_ref[...]
  xp = lax.dot_general(xb, w_ref[...], (((1,), (1,)), ((), ())),
                       preferred_element_type=jnp.float32)
  a_s = jnp.dot(xp, as_ref[...], preferred_element_type=jnp.float32)
  a_d = jnp.dot(xp, ad_ref[...], preferred_element_type=jnp.float32)
  t = a_s + a_d
  s = jnp.maximum(t, 0.2 * t)
  rows = xb.shape[0]
  st_ref[...] = jnp.concatenate(
      [a_s, jnp.zeros((rows, _OFF - _H), jnp.float32), xp], axis=1)
  dt_ref[...] = jnp.concatenate(
      [a_d, s, jnp.zeros((rows, _DD - 2 * _H), jnp.float32)], axis=1)


def _node_tables(x, w, a_src_m, a_dst_m):
  blk = 400
  grid = _N // blk
  return pl.pallas_call(
      _node_tables_body,
      grid=(grid,),
      in_specs=[
          pl.BlockSpec((blk, _IN), lambda i: (i, 0)),
          pl.BlockSpec((_HC, _IN), lambda i: (0, 0)),
          pl.BlockSpec((_HC, _H), lambda i: (0, 0)),
          pl.BlockSpec((_HC, _H), lambda i: (0, 0)),
      ],
      out_specs=[
          pl.BlockSpec((blk, _D), lambda i: (i, 0)),
          pl.BlockSpec((blk, _DD), lambda i: (i, 0)),
      ],
      out_shape=[
          jax.ShapeDtypeStruct((_N, _D), jnp.float32),
          jax.ShapeDtypeStruct((_N, _DD), jnp.float32),
      ],
  )(x, w, a_src_m, a_dst_m)


def _sc_edge_body(src_tab, dst_tab, ei_src, ei_dst, zeros_tab, acc_out,
                  sidx_all, didx_all, srcbuf, dstbuf, acc,
                  gsem0, gsem1, gsem2, gsem3, ssem0, ssem1, ssem2, ssem3):
  cid = lax.axis_index("c")
  sid = lax.axis_index("s")
  wid = sid * _NC + cid

  @pl.when(sid < _NSTRIPE)
  def _():
    pltpu.sync_copy(zeros_tab, acc.at[pl.ds(sid * _STRIPE, _STRIPE)])

  pltpu.sync_copy(ei_src.at[pl.ds(wid * _NCH, _NCH)], sidx_all)
  pltpu.sync_copy(ei_dst.at[pl.ds(wid * _NCH, _NCH)], didx_all)
  plsc.subcore_barrier()

  iota16 = lax.iota(jnp.int32, 16)

  def step(th):
    return jnp.minimum(jnp.maximum(iota16 - (th - 1), 0), 1)

  gidx = [iota16 * 0, step(8), 1 + step(2) + step(12), 3 + step(6)]
  qoff = (0, 16, 32, 48)
  sidx_shift = jnp.minimum(iota16 + _H, 2 * _H - 1)
  m_lt5 = (1 - step(_H)).astype(jnp.float32)
  m_geoff = step(_OFF).astype(jnp.float32)

  def _tga(v, idx):
    return jnp.take_along_axis(v, idx, axis=0, mode="promise_in_bounds")

  gsems = (gsem0, gsem1, gsem2, gsem3)
  ssems = (ssem0, ssem1, ssem2, ssem3)

  def issue_gathers(b, ch):
    pltpu.async_copy(src_tab.at[sidx_all.at[ch]], srcbuf.at[b], gsems[b])
    pltpu.async_copy(dst_tab.at[didx_all.at[ch]], dstbuf.at[b], gsems[b])

  def wait_gathers(b, ch):
    pltpu.make_async_copy(src_tab.at[sidx_all.at[ch]], srcbuf.at[b],
                          gsems[b]).wait()
    pltpu.make_async_copy(dst_tab.at[didx_all.at[ch]], dstbuf.at[b],
                          gsems[b]).wait()

  def issue_scatter(b, ch):
    pltpu.async_copy(srcbuf.at[b], acc.at[didx_all.at[ch]], ssems[b],
                     add=True)

  def wait_scatter(b, ch):
    pltpu.make_async_copy(srcbuf.at[b], acc.at[didx_all.at[ch]],
                          ssems[b]).wait()

  def compute(b):
    @plsc.parallel_loop(0, _B, unroll=8)
    def _(e):
      vd = dstbuf[b, e, :]
      vs = [srcbuf[b, e, pl.ds(q, 16)] for q in qoff]
      alpha = vs[0] + vd
      lr = jnp.maximum(alpha, 0.2 * alpha)
      exv = jnp.exp(lr - _tga(vd, sidx_shift))
      srcbuf[b, e, pl.ds(0, 16)] = (
          exv * m_lt5 + vs[0] * _tga(exv, gidx[0]) * m_geoff)
      for k in range(1, 4):
        srcbuf[b, e, pl.ds(qoff[k], 16)] = vs[k] * _tga(exv, gidx[k])

  issue_gathers(0, 0)
  issue_gathers(1, 1)

  def quad(i, carry):
    for b in range(4):
      ch = 4 * i + b
      wait_gathers(b, ch)
      compute(b)
      issue_scatter(b, ch)
      nb = (b + 2) % 4
      if b < 2:
        @pl.when(ch >= 2)
        def _():
          wait_scatter(nb, ch)
      else:
        wait_scatter(nb, ch)

      @pl.when(ch + 2 < _NCH)
      def _():
        issue_gathers(nb, ch + 2)
    return carry

  lax.fori_loop(0, _NCH // 4, quad, 0)
  wait_scatter(2, _NCH - 2)
  wait_scatter(3, _NCH - 1)

  plsc.subcore_barrier()

  @pl.when(sid < _NSTRIPE)
  def _():
    pltpu.sync_copy(
        acc.at[pl.ds(sid * _STRIPE, _STRIPE)],
        acc_out.at[pl.ds(cid * _N + sid * _STRIPE, _STRIPE)])


def _sc_edge(src_tab, dst_tab, ei_src, ei_dst, zeros_tab):
  mesh = plsc.VectorSubcoreMesh(core_axis_name="c", subcore_axis_name="s")
  fn = pl.kernel(
      _sc_edge_body,
      out_type=jax.ShapeDtypeStruct((_NC * _N, _D), jnp.float32),
      mesh=mesh,
      compiler_params=pltpu.CompilerParams(use_tc_tiling_on_sc=False),
      scratch_types=[
          pltpu.VMEM((_NCH, _B), jnp.int32),
          pltpu.VMEM((_NCH, _B), jnp.int32),
          pltpu.VMEM((4, _B, _D), jnp.float32),
          pltpu.VMEM((4, _B, _DD), jnp.float32),
          pltpu.VMEM_SHARED((_N, _D), jnp.float32),
          pltpu.SemaphoreType.DMA,
          pltpu.SemaphoreType.DMA,
          pltpu.SemaphoreType.DMA,
          pltpu.SemaphoreType.DMA,
          pltpu.SemaphoreType.DMA,
          pltpu.SemaphoreType.DMA,
          pltpu.SemaphoreType.DMA,
          pltpu.SemaphoreType.DMA,
      ],
  )
  return fn(src_tab, dst_tab, ei_src.reshape(_NW * _NCH, _B),
            ei_dst.reshape(_NW * _NCH, _B), zeros_tab)


def _finish_body(acc_ref, st_ref, batch_ref, bias_ref, linw_ref, linb_ref,
                 epx_ref, h_ref, out_ref):
  acc = acc_ref[pl.ds(0, _N), :] + acc_ref[pl.ds(_N, _N), :]
  st = st_ref[...]
  col = lax.broadcasted_iota(jnp.int32, (_N, _D), 1)
  self_row = jnp.where(col < _H, 1.0, jnp.where(col < _OFF, 0.0, st))
  tot = acc + self_row
  den = tot[:, 0:_H]
  num = tot[:, _OFF:_OFF + _HC]
  den50 = jnp.dot(den, epx_ref[...], preferred_element_type=jnp.float32)
  p = num / (den50 + 1e-16) + bias_ref[...]
  h1 = jnp.where(p > 0, p, jnp.exp(jnp.minimum(p, 0.0)) - 1.0)
  b = batch_ref[...]
  gids = lax.broadcasted_iota(jnp.int32, (_G, _N), 0)
  onehot = (gids == b).astype(jnp.float32)
  hsum = jnp.dot(onehot, h1, preferred_element_type=jnp.float32)
  cnt = jnp.sum(onehot, axis=1, keepdims=True)
  pooled = hsum / jnp.maximum(cnt, 1.0)
  logits = jnp.sum(pooled * linw_ref[...], axis=1, keepdims=True) + linb_ref[0, 0]
  h_ref[...] = pooled
  out_ref[...] = 1.0 / (1.0 + jnp.exp(-logits))


def _finish(acc2, src_tab, batch2, bias2, lin_w, lin_b2, epx):
  return pl.pallas_call(
      _finish_body,
      out_shape=[
          jax.ShapeDtypeStruct((_G, _HC), jnp.float32),
          jax.ShapeDtypeStruct((_G, 1), jnp.float32),
      ],
  )(acc2, src_tab, batch2, bias2, lin_w, lin_b2, epx)


def kernel(x, edge_index, batch, W, att_src, att_dst, bias, lin_w, lin_b):
  eye = jnp.eye(_H, dtype=jnp.float32)
  a_src_m = (att_src[:, :, None] * eye[:, None, :]).reshape(_HC, _H)
  a_dst_m = (att_dst[:, :, None] * eye[:, None, :]).reshape(_HC, _H)
  epx = jnp.repeat(eye, _C, axis=1)

  src_tab, dst_tab = _node_tables(x, W, a_src_m, a_dst_m)

  zeros_tab = jnp.zeros((_STRIPE, _D), jnp.float32)
  acc2 = _sc_edge(src_tab, dst_tab, edge_index[0], edge_index[1], zeros_tab)

  h, out = _finish(acc2, src_tab, batch.reshape(1, _N),
                   bias.reshape(1, _HC), lin_w, lin_b.reshape(1, 1), epx)
  return (h, out)


# speedup vs baseline: 1.0211x; 1.0211x over previous
import functools

import jax
import jax.numpy as jnp
from jax import lax
from jax.experimental import pallas as pl
from jax.experimental.pallas import tpu as pltpu
from jax.experimental.pallas import tpu_sc as plsc

_N = 10000
_E = 320000
_IN = 200
_H = 5
_C = 10
_HC = 50
_G = 16
_D = 64
_DD = 16
_OFF = 14
_NC = 2
_NS = 16
_NW = _NC * _NS
_EPT = _E // _NW
_B = 100
_NCH = _EPT // _B
_STRIPE = 1000
_NSTRIPE = _N // _STRIPE


def _node_tables_body(x_ref, w_ref, as_ref, ad_ref, st_ref, dt_ref):
  xb = x_ref[...]
  xp = lax.dot_general(xb, w_ref[...], (((1,), (1,)), ((), ())),
                       preferred_element_type=jnp.float32)
  a_s = jnp.dot(xp, as_ref[...], preferred_element_type=jnp.float32)
  a_d = jnp.dot(xp, ad_ref[...], preferred_element_type=jnp.float32)
  t = a_s + a_d
  s = jnp.maximum(t, 0.2 * t)
  rows = xb.shape[0]
  st_ref[...] = jnp.concatenate(
      [a_s, jnp.zeros((rows, _OFF - _H), jnp.float32), xp], axis=1)
  dt_ref[...] = jnp.concatenate(
      [a_d, s, jnp.zeros((rows, _DD - 2 * _H), jnp.float32)], axis=1)


def _node_tables(x, w, a_src_m, a_dst_m):
  blk = 400
  grid = _N // blk
  return pl.pallas_call(
      _node_tables_body,
      grid=(grid,),
      in_specs=[
          pl.BlockSpec((blk, _IN), lambda i: (i, 0)),
          pl.BlockSpec((_HC, _IN), lambda i: (0, 0)),
          pl.BlockSpec((_HC, _H), lambda i: (0, 0)),
          pl.BlockSpec((_HC, _H), lambda i: (0, 0)),
      ],
      out_specs=[
          pl.BlockSpec((blk, _D), lambda i: (i, 0)),
          pl.BlockSpec((blk, _DD), lambda i: (i, 0)),
      ],
      out_shape=[
          jax.ShapeDtypeStruct((_N, _D), jnp.float32),
          jax.ShapeDtypeStruct((_N, _DD), jnp.float32),
      ],
  )(x, w, a_src_m, a_dst_m)


def _sc_edge_body(src_tab, dst_tab, ei_src, ei_dst, zeros_tab, acc_out,
                  sidx_all, didx_all, srcbuf, dstbuf, acc,
                  gsem0, gsem1, gsem2, gsem3, ssem0, ssem1, ssem2, ssem3):
  cid = lax.axis_index("c")
  sid = lax.axis_index("s")
  wid = sid * _NC + cid

  @pl.when(sid < _NSTRIPE)
  def _():
    pltpu.sync_copy(zeros_tab, acc.at[pl.ds(sid * _STRIPE, _STRIPE)])

  pltpu.sync_copy(ei_src.at[pl.ds(wid * _NCH, _NCH)], sidx_all)
  pltpu.sync_copy(ei_dst.at[pl.ds(wid * _NCH, _NCH)], didx_all)
  plsc.subcore_barrier()

  iota16 = lax.iota(jnp.int32, 16)

  def step(th):
    return jnp.minimum(jnp.maximum(iota16 - (th - 1), 0), 1)

  gidx = [iota16 * 0, step(8), 1 + step(2) + step(12), 3 + step(6)]
  qoff = (0, 16, 32, 48)
  sidx_shift = jnp.minimum(iota16 + _H, 2 * _H - 1)
  m_lt5 = (1 - step(_H)).astype(jnp.float32)
  m_geoff = step(_OFF).astype(jnp.float32)

  def _tga(v, idx):
    return jnp.take_along_axis(v, idx, axis=0, mode="promise_in_bounds")

  gsems = (gsem0, gsem1, gsem2, gsem3)
  ssems = (ssem0, ssem1, ssem2, ssem3)

  def issue_gathers(b, ch):
    pltpu.async_copy(src_tab.at[sidx_all.at[ch]], srcbuf.at[b], gsems[b])
    pltpu.async_copy(dst_tab.at[didx_all.at[ch]], dstbuf.at[b], gsems[b])

  def wait_gathers(b, ch):
    pltpu.make_async_copy(src_tab.at[sidx_all.at[ch]], srcbuf.at[b],
                          gsems[b]).wait()
    pltpu.make_async_copy(dst_tab.at[didx_all.at[ch]], dstbuf.at[b],
                          gsems[b]).wait()

  def issue_scatter(b, ch):
    pltpu.async_copy(srcbuf.at[b], acc.at[didx_all.at[ch]], ssems[b],
                     add=True)

  def wait_scatter(b, ch):
    pltpu.make_async_copy(srcbuf.at[b], acc.at[didx_all.at[ch]],
                          ssems[b]).wait()

  def compute(b):
    @plsc.parallel_loop(0, _B, unroll=4)
    def _(e):
      vd = dstbuf[b, e, :]
      vs = [srcbuf[b, e, pl.ds(q, 16)] for q in qoff]
      alpha = vs[0] + vd
      lr = jnp.maximum(alpha, 0.2 * alpha)
      exv = jnp.exp(lr - _tga(vd, sidx_shift))
      srcbuf[b, e, pl.ds(0, 16)] = (
          exv * m_lt5 + vs[0] * _tga(exv, gidx[0]) * m_geoff)
      for k in range(1, 4):
        srcbuf[b, e, pl.ds(qoff[k], 16)] = vs[k] * _tga(exv, gidx[k])

  issue_gathers(0, 0)
  issue_gathers(1, 1)

  def quad(i, carry):
    for b in range(4):
      ch = 4 * i + b
      wait_gathers(b, ch)
      compute(b)
      issue_scatter(b, ch)
      nb = (b + 2) % 4
      if b < 2:
        @pl.when(ch >= 2)
        def _():
          wait_scatter(nb, ch)
      else:
        wait_scatter(nb, ch)

      @pl.when(ch + 2 < _NCH)
      def _():
        issue_gathers(nb, ch + 2)
    return carry

  lax.fori_loop(0, _NCH // 4, quad, 0)
  wait_scatter(2, _NCH - 2)
  wait_scatter(3, _NCH - 1)

  plsc.subcore_barrier()

  @pl.when(sid < _NSTRIPE)
  def _():
    pltpu.sync_copy(
        acc.at[pl.ds(sid * _STRIPE, _STRIPE)],
        acc_out.at[pl.ds(cid * _N + sid * _STRIPE, _STRIPE)])


def _sc_edge(src_tab, dst_tab, ei_src, ei_dst, zeros_tab):
  mesh = plsc.VectorSubcoreMesh(core_axis_name="c", subcore_axis_name="s")
  fn = pl.kernel(
      _sc_edge_body,
      out_type=jax.ShapeDtypeStruct((_NC * _N, _D), jnp.float32),
      mesh=mesh,
      compiler_params=pltpu.CompilerParams(use_tc_tiling_on_sc=False),
      scratch_types=[
          pltpu.VMEM((_NCH, _B), jnp.int32),
          pltpu.VMEM((_NCH, _B), jnp.int32),
          pltpu.VMEM((4, _B, _D), jnp.float32),
          pltpu.VMEM((4, _B, _DD), jnp.float32),
          pltpu.VMEM_SHARED((_N, _D), jnp.float32),
          pltpu.SemaphoreType.DMA,
          pltpu.SemaphoreType.DMA,
          pltpu.SemaphoreType.DMA,
          pltpu.SemaphoreType.DMA,
          pltpu.SemaphoreType.DMA,
          pltpu.SemaphoreType.DMA,
          pltpu.SemaphoreType.DMA,
          pltpu.SemaphoreType.DMA,
      ],
  )
  return fn(src_tab, dst_tab, ei_src.reshape(_NW * _NCH, _B),
            ei_dst.reshape(_NW * _NCH, _B), zeros_tab)


def _finish_body(acc_ref, st_ref, batch_ref, bias_ref, linw_ref, linb_ref,
                 epx_ref, h_ref, out_ref):
  acc = acc_ref[pl.ds(0, _N), :] + acc_ref[pl.ds(_N, _N), :]
  st = st_ref[...]
  col = lax.broadcasted_iota(jnp.int32, (_N, _D), 1)
  self_row = jnp.where(col < _H, 1.0, jnp.where(col < _OFF, 0.0, st))
  tot = acc + self_row
  den = tot[:, 0:_H]
  num = tot[:, _OFF:_OFF + _HC]
  den50 = jnp.dot(den, epx_ref[...], preferred_element_type=jnp.float32)
  p = num / (den50 + 1e-16) + bias_ref[...]
  h1 = jnp.where(p > 0, p, jnp.exp(jnp.minimum(p, 0.0)) - 1.0)
  b = batch_ref[...]
  gids = lax.broadcasted_iota(jnp.int32, (_G, _N), 0)
  onehot = (gids == b).astype(jnp.float32)
  hsum = jnp.dot(onehot, h1, preferred_element_type=jnp.float32)
  cnt = jnp.sum(onehot, axis=1, keepdims=True)
  pooled = hsum / jnp.maximum(cnt, 1.0)
  logits = jnp.sum(pooled * linw_ref[...], axis=1, keepdims=True) + linb_ref[0, 0]
  h_ref[...] = pooled
  out_ref[...] = 1.0 / (1.0 + jnp.exp(-logits))


def _finish(acc2, src_tab, batch2, bias2, lin_w, lin_b2, epx):
  return pl.pallas_call(
      _finish_body,
      out_shape=[
          jax.ShapeDtypeStruct((_G, _HC), jnp.float32),
          jax.ShapeDtypeStruct((_G, 1), jnp.float32),
      ],
  )(acc2, src_tab, batch2, bias2, lin_w, lin_b2, epx)


def kernel(x, edge_index, batch, W, att_src, att_dst, bias, lin_w, lin_b):
  eye = jnp.eye(_H, dtype=jnp.float32)
  a_src_m = (att_src[:, :, None] * eye[:, None, :]).reshape(_HC, _H)
  a_dst_m = (att_dst[:, :, None] * eye[:, None, :]).reshape(_HC, _H)
  epx = jnp.repeat(eye, _C, axis=1)

  src_tab, dst_tab = _node_tables(x, W, a_src_m, a_dst_m)

  zeros_tab = jnp.zeros((_STRIPE, _D), jnp.float32)
  acc2 = _sc_edge(src_tab, dst_tab, edge_index[0], edge_index[1], zeros_tab)

  h, out = _finish(acc2, src_tab, batch.reshape(1, _N),
                   bias.reshape(1, _HC), lin_w, lin_b.reshape(1, 1), epx)
  return (h, out)
